# baseline (speedup 1.0000x reference)
import functools

import jax
import jax.numpy as jnp
from jax import lax
from jax.experimental import pallas as pl
from jax.experimental.pallas import tpu as pltpu
from jax.experimental.pallas import tpu_sc as plsc

DIM = 2048
HID = 16384
K = 16
N = 32
BH = 2048
NBLK = HID // BH

DSPLIT = 1024
DREM = DIM - DSPLIT
RPK = DREM // 128

L = 16

_NEG = jnp.finfo(jnp.float32).min


def _encode_topk_body(x_ref, db_ref, ew_ref, eb_ref,
                      h_ref, vals_ref, idx_ref, rv_ref, ri_ref):
    i = pl.program_id(0)
    xc = x_ref[...] - db_ref[...]
    h = jax.lax.dot_general(xc, ew_ref[...], (((1,), (1,)), ((), ())),
                            preferred_element_type=jnp.float32) + eb_ref[...]
    h_ref[...] = h

    @pl.when(i == 0)
    def _init():
        rv_ref[...] = jnp.full((N, K), _NEG, jnp.float32)
        ri_ref[...] = jnp.zeros((N, K), jnp.int32)

    col = jax.lax.broadcasted_iota(jnp.int32, (N, BH), 1)
    work = h
    bv = []
    bi = []
    for _ in range(K):
        m = jnp.max(work, axis=1, keepdims=True)
        first = jnp.min(jnp.where(work == m, col, BH), axis=1, keepdims=True)
        bv.append(m)
        bi.append(first + i * BH)
        work = jnp.where(col == first, _NEG, work)
    cand_v = jnp.concatenate([rv_ref[...]] + bv, axis=1)
    cand_i = jnp.concatenate([ri_ref[...]] + bi, axis=1)
    ccol = jax.lax.broadcasted_iota(jnp.int32, (N, 2 * K), 1)
    nrv = []
    nri = []
    for _ in range(K):
        m = jnp.max(cand_v, axis=1, keepdims=True)
        pos = jnp.min(jnp.where(cand_v == m, ccol, 2 * K), axis=1, keepdims=True)
        sel = ccol == pos
        nrv.append(m)
        nri.append(jnp.min(jnp.where(sel, cand_i, HID), axis=1, keepdims=True))
        cand_v = jnp.where(sel, _NEG, cand_v)
    rv_ref[...] = jnp.concatenate(nrv, axis=1)
    ri_ref[...] = jnp.concatenate(nri, axis=1)

    @pl.when(i == NBLK - 1)
    def _emit():
        vals_ref[...] = jnp.maximum(rv_ref[...], 0.0)
        idx_ref[...] = ri_ref[...]


def _decode_tc_body(h_ref, idx_ref, dw_ref, db_ref, out_ref):
    i = pl.program_id(0)
    col = jax.lax.broadcasted_iota(jnp.int32, (N, BH), 1) + i * BH
    keep = jnp.zeros((N, BH), jnp.bool_)
    for k in range(K):
        keep = jnp.logical_or(keep, col == idx_ref[:, k:k + 1])
    hs = jnp.maximum(jnp.where(keep, h_ref[...], 0.0), 0.0)
    part = jax.lax.dot_general(hs, dw_ref[...], (((1,), (1,)), ((), ())),
                               preferred_element_type=jnp.float32)

    @pl.when(i == 0)
    def _init():
        out_ref[...] = part + db_ref[...]

    @pl.when(i != 0)
    def _acc():
        out_ref[...] += part


def _sc_body(vals_hbm, idx_hbm, dw_hbm, db_hbm, out_hbm,
             vals_v, idxs_v, d_v, idx_v, g_v, out_v, sem):
    n = lax.axis_index("s") * 2 + lax.axis_index("c")
    pltpu.sync_copy(vals_hbm.at[n], vals_v)
    pltpu.sync_copy(idx_hbm.at[n], idxs_v)
    pltpu.sync_copy(db_hbm.at[pl.ds(DSPLIT, DREM)], out_v)

    iota = lax.iota(jnp.int32, L)
    vals = vals_v[pl.ds(0, L)]
    idxs = idxs_v[pl.ds(0, L)]

    def build_d(c, _):
        d = DSPLIT + c * L + iota
        d_v[pl.ds(c * L, L)] = (d >> 3) * (128 * 1024) + (d & 7) * 128
        return 0

    lax.fori_loop(0, DREM // L, build_d, 0)

    def build_k(k, _):
        jk = jnp.max(jnp.where(iota == k, idxs, -1))
        cj = (jk >> 7) * 1024 + (jk & 127)
        for t in range(RPK):
            r = k * RPK + t
            for l in range(8):
                idx_v[r, pl.ds(l * L, L)] = d_v[pl.ds(t * 128 + l * L, L)] + cj
            pltpu.async_copy(dw_hbm.at[idx_v.at[r]], g_v.at[r], sem)
        return 0

    lax.fori_loop(0, K, build_k, 0)

    def drain_k(k, _):
        for t in range(RPK):
            r = k * RPK + t
            pltpu.make_async_copy(dw_hbm.at[idx_v.at[r]], g_v.at[r], sem).wait()
        return 0

    lax.fori_loop(0, K, drain_k, 0)

    vks = [jnp.max(jnp.where(iota == k, vals, -1.0)) for k in range(K)]

    def dec_blk(c, _):
        acc = out_v[pl.ds(c * L, L)]
        t = c >> 3
        sl = pl.ds((c & 7) * L, L)
        for k in range(K):
            acc = acc + vks[k] * g_v[k * RPK + t, sl]
        out_v[pl.ds(c * L, L)] = acc
        return 0

    lax.fori_loop(0, DREM // L, dec_blk, 0)

    pltpu.sync_copy(out_v, out_hbm.at[n])


def kernel(x, enc_W, enc_b, dec_W, dec_b):
    h, vals, idxv = pl.pallas_call(
        _encode_topk_body,
        grid=(NBLK,),
        in_specs=[
            pl.BlockSpec((N, DIM), lambda i: (0, 0)),
            pl.BlockSpec((DIM,), lambda i: (0,)),
            pl.BlockSpec((BH, DIM), lambda i: (i, 0)),
            pl.BlockSpec((BH,), lambda i: (i,)),
        ],
        out_specs=[
            pl.BlockSpec((N, BH), lambda i: (0, i)),
            pl.BlockSpec((N, K), lambda i: (0, 0)),
            pl.BlockSpec((N, K), lambda i: (0, 0)),
        ],
        out_shape=[
            jax.ShapeDtypeStruct((N, HID), jnp.float32),
            jax.ShapeDtypeStruct((N, K), jnp.float32),
            jax.ShapeDtypeStruct((N, K), jnp.int32),
        ],
        scratch_shapes=[
            pltpu.VMEM((N, K), jnp.float32),
            pltpu.VMEM((N, K), jnp.int32),
        ],
    )(x, dec_b, enc_W, enc_b)

    mesh = plsc.VectorSubcoreMesh(core_axis_name="c", subcore_axis_name="s")
    sc = functools.partial(
        pl.kernel,
        mesh=mesh,
        compiler_params=pltpu.CompilerParams(needs_layout_passes=False),
        out_type=jax.ShapeDtypeStruct((N, DREM), jnp.float32),
        scratch_types=[
            pltpu.VMEM((K,), jnp.float32),
            pltpu.VMEM((K,), jnp.int32),
            pltpu.VMEM((DREM,), jnp.int32),
            pltpu.VMEM((K * RPK, 128), jnp.int32),
            pltpu.VMEM((K * RPK, 128), jnp.float32),
            pltpu.VMEM((DREM,), jnp.float32),
            pltpu.SemaphoreType.DMA,
        ],
    )(_sc_body)
    dwp = (dec_W.reshape(DIM // 8, 8, HID // 128, 128)
           .transpose(0, 2, 1, 3).reshape(DIM * HID))
    out_sc = sc(vals, idxv, dwp, dec_b)

    out_tc = pl.pallas_call(
        _decode_tc_body,
        grid=(NBLK,),
        in_specs=[
            pl.BlockSpec((N, BH), lambda i: (0, i)),
            pl.BlockSpec((N, K), lambda i: (0, 0)),
            pl.BlockSpec((DSPLIT, BH), lambda i: (0, i)),
            pl.BlockSpec((DSPLIT,), lambda i: (0,)),
        ],
        out_specs=pl.BlockSpec((N, DSPLIT), lambda i: (0, 0)),
        out_shape=jax.ShapeDtypeStruct((N, DSPLIT), jnp.float32),
    )(h, idxv, dec_W, dec_b[:DSPLIT])
    return jnp.concatenate([out_tc, out_sc], axis=1)


# speedup vs baseline: 1.3247x; 1.3247x over previous
import functools

import jax
import jax.numpy as jnp
from jax import lax
from jax.experimental import pallas as pl
from jax.experimental.pallas import tpu as pltpu
from jax.experimental.pallas import tpu_sc as plsc

DIM = 2048
HID = 16384
K = 16
N = 32
BH = 2048
NBLK = HID // BH

DSPLIT = 1024
DREM = DIM - DSPLIT
RPK = DREM // 128

L = 16

_NEG = jnp.finfo(jnp.float32).min


def _encode_topk_body(x_ref, db_ref, ew_ref, eb_ref,
                      h_ref, vals_ref, idx_ref, hacc_ref):
    i = pl.program_id(0)
    xc = x_ref[...] - db_ref[...]
    h = jax.lax.dot_general(xc, ew_ref[...], (((1,), (1,)), ((), ())),
                            preferred_element_type=jnp.float32) + eb_ref[...]
    h_ref[...] = h
    hacc_ref[:, pl.ds(i * BH, BH)] = h

    @pl.when(i == NBLK - 1)
    def _topk():
        col = jax.lax.broadcasted_iota(jnp.int32, (N, HID), 1)
        work = hacc_ref[...]
        for k in range(K):
            m = jnp.max(work, axis=1, keepdims=True)
            first = jnp.min(jnp.where(work == m, col, HID), axis=1,
                            keepdims=True)
            work = jnp.where(col == first, _NEG, work)
            vals_ref[:, k:k + 1] = jnp.maximum(m, 0.0)
            idx_ref[:, k:k + 1] = first


def _decode_tc_body(h_ref, idx_ref, dw_ref, db_ref, out_ref):
    i = pl.program_id(0)
    col = jax.lax.broadcasted_iota(jnp.int32, (N, BH), 1) + i * BH
    keep = jnp.zeros((N, BH), jnp.bool_)
    for k in range(K):
        keep = jnp.logical_or(keep, col == idx_ref[:, k:k + 1])
    hs = jnp.maximum(jnp.where(keep, h_ref[...], 0.0), 0.0)
    part = jax.lax.dot_general(hs, dw_ref[...], (((1,), (1,)), ((), ())),
                               preferred_element_type=jnp.float32)

    @pl.when(i == 0)
    def _init():
        out_ref[...] = part + db_ref[...]

    @pl.when(i != 0)
    def _acc():
        out_ref[...] += part


def _sc_body(vals_hbm, idx_hbm, dw_hbm, db_hbm, out_hbm,
             vals_v, idxs_v, d_v, idx_v, g_v, out_v, sem):
    n = lax.axis_index("s") * 2 + lax.axis_index("c")
    pltpu.sync_copy(vals_hbm.at[n], vals_v)
    pltpu.sync_copy(idx_hbm.at[n], idxs_v)
    pltpu.sync_copy(db_hbm.at[pl.ds(DSPLIT, DREM)], out_v)

    iota = lax.iota(jnp.int32, L)
    vals = vals_v[pl.ds(0, L)]
    idxs = idxs_v[pl.ds(0, L)]

    def build_d(c, _):
        d = DSPLIT + c * L + iota
        d_v[pl.ds(c * L, L)] = (d >> 3) * (128 * 1024) + (d & 7) * 128
        return 0

    lax.fori_loop(0, DREM // L, build_d, 0)

    def build_k(k, _):
        jk = jnp.max(jnp.where(iota == k, idxs, -1))
        cj = (jk >> 7) * 1024 + (jk & 127)
        for t in range(RPK):
            r = k * RPK + t
            for l in range(8):
                idx_v[r, pl.ds(l * L, L)] = d_v[pl.ds(t * 128 + l * L, L)] + cj
            pltpu.async_copy(dw_hbm.at[idx_v.at[r]], g_v.at[r], sem)
        return 0

    lax.fori_loop(0, K, build_k, 0)

    def drain_k(k, _):
        for t in range(RPK):
            r = k * RPK + t
            pltpu.make_async_copy(dw_hbm.at[idx_v.at[r]], g_v.at[r], sem).wait()
        return 0

    lax.fori_loop(0, K, drain_k, 0)

    vks = [jnp.max(jnp.where(iota == k, vals, -1.0)) for k in range(K)]

    def dec_blk(c, _):
        acc = out_v[pl.ds(c * L, L)]
        t = c >> 3
        sl = pl.ds((c & 7) * L, L)
        for k in range(K):
            acc = acc + vks[k] * g_v[k * RPK + t, sl]
        out_v[pl.ds(c * L, L)] = acc
        return 0

    lax.fori_loop(0, DREM // L, dec_blk, 0)

    pltpu.sync_copy(out_v, out_hbm.at[n])


def kernel(x, enc_W, enc_b, dec_W, dec_b):
    h, vals, idxv = pl.pallas_call(
        _encode_topk_body,
        grid=(NBLK,),
        in_specs=[
            pl.BlockSpec((N, DIM), lambda i: (0, 0)),
            pl.BlockSpec((DIM,), lambda i: (0,)),
            pl.BlockSpec((BH, DIM), lambda i: (i, 0)),
            pl.BlockSpec((BH,), lambda i: (i,)),
        ],
        out_specs=[
            pl.BlockSpec((N, BH), lambda i: (0, i)),
            pl.BlockSpec((N, K), lambda i: (0, 0)),
            pl.BlockSpec((N, K), lambda i: (0, 0)),
        ],
        out_shape=[
            jax.ShapeDtypeStruct((N, HID), jnp.float32),
            jax.ShapeDtypeStruct((N, K), jnp.float32),
            jax.ShapeDtypeStruct((N, K), jnp.int32),
        ],
        scratch_shapes=[
            pltpu.VMEM((N, HID), jnp.float32),
        ],
    )(x, dec_b, enc_W, enc_b)

    mesh = plsc.VectorSubcoreMesh(core_axis_name="c", subcore_axis_name="s")
    sc = functools.partial(
        pl.kernel,
        mesh=mesh,
        compiler_params=pltpu.CompilerParams(needs_layout_passes=False),
        out_type=jax.ShapeDtypeStruct((N, DREM), jnp.float32),
        scratch_types=[
            pltpu.VMEM((K,), jnp.float32),
            pltpu.VMEM((K,), jnp.int32),
            pltpu.VMEM((DREM,), jnp.int32),
            pltpu.VMEM((K * RPK, 128), jnp.int32),
            pltpu.VMEM((K * RPK, 128), jnp.float32),
            pltpu.VMEM((DREM,), jnp.float32),
            pltpu.SemaphoreType.DMA,
        ],
    )(_sc_body)
    dwp = (dec_W.reshape(DIM // 8, 8, HID // 128, 128)
           .transpose(0, 2, 1, 3).reshape(DIM * HID))
    out_sc = sc(vals, idxv, dwp, dec_b)

    out_tc = pl.pallas_call(
        _decode_tc_body,
        grid=(NBLK,),
        in_specs=[
            pl.BlockSpec((N, BH), lambda i: (0, i)),
            pl.BlockSpec((N, K), lambda i: (0, 0)),
            pl.BlockSpec((DSPLIT, BH), lambda i: (0, i)),
            pl.BlockSpec((DSPLIT,), lambda i: (0,)),
        ],
        out_specs=pl.BlockSpec((N, DSPLIT), lambda i: (0, 0)),
        out_shape=jax.ShapeDtypeStruct((N, DSPLIT), jnp.float32),
    )(h, idxv, dec_W, dec_b[:DSPLIT])
    return jnp.concatenate([out_tc, out_sc], axis=1)


# speedup vs baseline: 1.3434x; 1.0141x over previous
import functools

import jax
import jax.numpy as jnp
from jax import lax
from jax.experimental import pallas as pl
from jax.experimental.pallas import tpu as pltpu
from jax.experimental.pallas import tpu_sc as plsc

DIM = 2048
HID = 16384
K = 16
N = 32
BH = 2048
NBLK = HID // BH

DSPLIT = 1152
DREM = DIM - DSPLIT
RPK = DREM // 128

L = 16

_NEG = jnp.finfo(jnp.float32).min


def _encode_topk_body(x_ref, db_ref, ew_ref, eb_ref,
                      h_ref, vals_ref, idx_ref, hacc_ref):
    i = pl.program_id(0)
    xc = x_ref[...] - db_ref[...]
    h = jax.lax.dot_general(xc, ew_ref[...], (((1,), (1,)), ((), ())),
                            preferred_element_type=jnp.float32) + eb_ref[...]
    h_ref[...] = h
    hacc_ref[:, pl.ds(i * BH, BH)] = h

    @pl.when(i == NBLK - 1)
    def _topk():
        col = jax.lax.broadcasted_iota(jnp.int32, (N, HID), 1)
        work = hacc_ref[...]
        for k in range(K):
            m = jnp.max(work, axis=1, keepdims=True)
            first = jnp.min(jnp.where(work == m, col, HID), axis=1,
                            keepdims=True)
            work = jnp.where(col == first, _NEG, work)
            vals_ref[:, k:k + 1] = jnp.maximum(m, 0.0)
            idx_ref[:, k:k + 1] = first


def _decode_tc_body(h_ref, idx_ref, dw_ref, db_ref, out_ref):
    i = pl.program_id(0)
    col = jax.lax.broadcasted_iota(jnp.int32, (N, BH), 1) + i * BH
    keep = jnp.zeros((N, BH), jnp.bool_)
    for k in range(K):
        keep = jnp.logical_or(keep, col == idx_ref[:, k:k + 1])
    hs = jnp.maximum(jnp.where(keep, h_ref[...], 0.0), 0.0)
    part = jax.lax.dot_general(hs, dw_ref[...], (((1,), (1,)), ((), ())),
                               preferred_element_type=jnp.float32)

    @pl.when(i == 0)
    def _init():
        out_ref[...] = part + db_ref[...]

    @pl.when(i != 0)
    def _acc():
        out_ref[...] += part


def _sc_body(vals_hbm, idx_hbm, dw_hbm, db_hbm, out_hbm,
             vals_v, idxs_v, d_v, idx_v, g_v, out_v, sem):
    n = lax.axis_index("s") * 2 + lax.axis_index("c")
    pltpu.sync_copy(vals_hbm.at[n], vals_v)
    pltpu.sync_copy(idx_hbm.at[n], idxs_v)
    pltpu.sync_copy(db_hbm.at[pl.ds(DSPLIT, DREM)], out_v)

    iota = lax.iota(jnp.int32, L)
    vals = vals_v[pl.ds(0, L)]
    idxs = idxs_v[pl.ds(0, L)]

    def build_d(c, _):
        d = DSPLIT + c * L + iota
        d_v[pl.ds(c * L, L)] = (d >> 3) * (128 * 1024) + (d & 7) * 128
        return 0

    lax.fori_loop(0, DREM // L, build_d, 0)

    def build_k(k, _):
        jk = jnp.max(jnp.where(iota == k, idxs, -1))
        cj = (jk >> 7) * 1024 + (jk & 127)
        for t in range(RPK):
            r = k * RPK + t
            for l in range(8):
                idx_v[r, pl.ds(l * L, L)] = d_v[pl.ds(t * 128 + l * L, L)] + cj
            pltpu.async_copy(dw_hbm.at[idx_v.at[r]], g_v.at[r], sem)
        return 0

    lax.fori_loop(0, K, build_k, 0)

    def drain_k(k, _):
        for t in range(RPK):
            r = k * RPK + t
            pltpu.make_async_copy(dw_hbm.at[idx_v.at[r]], g_v.at[r], sem).wait()
        return 0

    lax.fori_loop(0, K, drain_k, 0)

    vks = [jnp.max(jnp.where(iota == k, vals, -1.0)) for k in range(K)]

    def dec_blk(c, _):
        acc = out_v[pl.ds(c * L, L)]
        t = c >> 3
        sl = pl.ds((c & 7) * L, L)
        for k in range(K):
            acc = acc + vks[k] * g_v[k * RPK + t, sl]
        out_v[pl.ds(c * L, L)] = acc
        return 0

    lax.fori_loop(0, DREM // L, dec_blk, 0)

    pltpu.sync_copy(out_v, out_hbm.at[n])


def kernel(x, enc_W, enc_b, dec_W, dec_b):
    h, vals, idxv = pl.pallas_call(
        _encode_topk_body,
        grid=(NBLK,),
        in_specs=[
            pl.BlockSpec((N, DIM), lambda i: (0, 0)),
            pl.BlockSpec((DIM,), lambda i: (0,)),
            pl.BlockSpec((BH, DIM), lambda i: (i, 0)),
            pl.BlockSpec((BH,), lambda i: (i,)),
        ],
        out_specs=[
            pl.BlockSpec((N, BH), lambda i: (0, i)),
            pl.BlockSpec((N, K), lambda i: (0, 0)),
            pl.BlockSpec((N, K), lambda i: (0, 0)),
        ],
        out_shape=[
            jax.ShapeDtypeStruct((N, HID), jnp.float32),
            jax.ShapeDtypeStruct((N, K), jnp.float32),
            jax.ShapeDtypeStruct((N, K), jnp.int32),
        ],
        scratch_shapes=[
            pltpu.VMEM((N, HID), jnp.float32),
        ],
    )(x, dec_b, enc_W, enc_b)

    mesh = plsc.VectorSubcoreMesh(core_axis_name="c", subcore_axis_name="s")
    sc = functools.partial(
        pl.kernel,
        mesh=mesh,
        compiler_params=pltpu.CompilerParams(needs_layout_passes=False),
        out_type=jax.ShapeDtypeStruct((N, DREM), jnp.float32),
        scratch_types=[
            pltpu.VMEM((K,), jnp.float32),
            pltpu.VMEM((K,), jnp.int32),
            pltpu.VMEM((DREM,), jnp.int32),
            pltpu.VMEM((K * RPK, 128), jnp.int32),
            pltpu.VMEM((K * RPK, 128), jnp.float32),
            pltpu.VMEM((DREM,), jnp.float32),
            pltpu.SemaphoreType.DMA,
        ],
    )(_sc_body)
    dwp = (dec_W.reshape(DIM // 8, 8, HID // 128, 128)
           .transpose(0, 2, 1, 3).reshape(DIM * HID))
    out_sc = sc(vals, idxv, dwp, dec_b)

    out_tc = pl.pallas_call(
        _decode_tc_body,
        grid=(NBLK,),
        in_specs=[
            pl.BlockSpec((N, BH), lambda i: (0, i)),
            pl.BlockSpec((N, K), lambda i: (0, 0)),
            pl.BlockSpec((DSPLIT, BH), lambda i: (0, i)),
            pl.BlockSpec((DSPLIT,), lambda i: (0,)),
        ],
        out_specs=pl.BlockSpec((N, DSPLIT), lambda i: (0, 0)),
        out_shape=jax.ShapeDtypeStruct((N, DSPLIT), jnp.float32),
    )(h, idxv, dec_W, dec_b[:DSPLIT])
    return jnp.concatenate([out_tc, out_sc], axis=1)
